# Initial kernel scaffold; baseline (speedup 1.0000x reference)
import jax
import jax.numpy as jnp
from jax.experimental import pallas as pl


def kernel(pool, ind, k_size):
    raise NotImplementedError("write your pallas kernel here")


# speedup vs baseline: 3.9929x; 3.9929x over previous
import functools

import jax
import jax.numpy as jnp
from jax import lax
from jax.experimental import pallas as pl
from jax.experimental.pallas import tpu as pltpu
from jax.experimental.pallas import tpu_sc as plsc

B, H, W_IN, C = 1, 256, 256, 96
KS = 2
N = B * H * W_IN * C
OUT = (H * KS) * (W_IN * KS) * C
NC, NS, L = 2, 16, 16
NWIN = 32
WIN = OUT // NWIN
PASSES = NWIN // NC
CHUNK = 16384
SHARE = N // NS
NCHUNK = SHARE // CHUNK
WSLICE = WIN // NS
NWB = WSLICE // CHUNK


def _unpool_sc(ind_flat, pool_flat):
    mesh = plsc.VectorSubcoreMesh(core_axis_name="c", subcore_axis_name="s")

    @functools.partial(
        pl.kernel,
        mesh=mesh,
        out_type=jax.ShapeDtypeStruct((OUT,), jnp.float32),
        scratch_types=[
            pltpu.VMEM((CHUNK,), jnp.int32),
            pltpu.VMEM((CHUNK,), jnp.float32),
            pltpu.VMEM((CHUNK,), jnp.int32),
            pltpu.VMEM((CHUNK,), jnp.float32),
            pltpu.VMEM((CHUNK,), jnp.float32),
            pltpu.VMEM_SHARED((WIN,), jnp.float32),
        ],
    )
    def k(ind_hbm, pool_hbm, out_hbm,
          idx_v, val_v, tidx_v, tval_v, zero_v, win_sh):
        c = lax.axis_index("c")
        s = lax.axis_index("s")
        lanes = lax.iota(jnp.int32, L)

        def zb(i, carry):
            zero_v[pl.ds(i * L, L)] = jnp.zeros((L,), jnp.float32)
            return carry
        lax.fori_loop(0, CHUNK // L, zb, None)

        def do_pass(p, carry):
            lo = (p * NC + c) * WIN

            def zwin(j, cy):
                pltpu.sync_copy(zero_v,
                                win_sh.at[pl.ds(s * WSLICE + j * CHUNK, CHUNK)])
                return cy
            lax.fori_loop(0, NWB, zwin, None)
            plsc.subcore_barrier()

            def do_chunk(kk, cy):
                base = s * SHARE + kk * CHUNK
                pltpu.sync_copy(ind_hbm.at[pl.ds(base, CHUNK)], idx_v)
                pltpu.sync_copy(pool_hbm.at[pl.ds(base, CHUNK)], val_v)

                def vec(i, cz):
                    iv = idx_v[pl.ds(i * L, L)]
                    vv = val_v[pl.ds(i * L, L)]
                    rel = iv - lo
                    ok = (rel >= 0) & (rel < WIN)
                    pad = (s * CHUNK + i * L) + lanes
                    tidx_v[pl.ds(i * L, L)] = jnp.where(ok, rel, pad)
                    tval_v[pl.ds(i * L, L)] = jnp.where(ok, vv, 0.0)
                    return cz
                lax.fori_loop(0, CHUNK // L, vec, None)
                pltpu.sync_copy(tval_v, win_sh.at[tidx_v], add=True)
                return cy
            lax.fori_loop(0, NCHUNK, do_chunk, None)
            plsc.subcore_barrier()

            def wb(j, cy):
                o = s * WSLICE + j * CHUNK
                pltpu.sync_copy(win_sh.at[pl.ds(o, CHUNK)],
                                out_hbm.at[pl.ds(lo + o, CHUNK)])
                return cy
            lax.fori_loop(0, NWB, wb, None)
            plsc.subcore_barrier()
            return carry
        lax.fori_loop(0, PASSES, do_pass, None)

    return k(ind_flat, pool_flat)


def kernel(pool, ind, k_size):
    pool_flat = pool.reshape(N)
    ind_flat = ind.reshape(N) + (jnp.asarray(k_size, jnp.int32) - KS)
    out = _unpool_sc(ind_flat, pool_flat)
    return out.reshape(B, H * KS, W_IN * KS, C)
